# baseline (speedup 1.0000x reference)
import functools

import jax
import jax.numpy as jnp
from jax import lax
from jax.experimental import pallas as pl
from jax.experimental.pallas import tpu as pltpu
from jax.experimental.pallas import tpu_sc as plsc


def kernel(input, indices):
    B, C, H, W = input.shape
    num_cores = 2
    per_core = C // num_cores
    mesh = plsc.ScalarSubcoreMesh(axis_name="core", num_cores=num_cores)

    @functools.partial(
        pl.kernel,
        out_type=jax.ShapeDtypeStruct(input.shape, input.dtype),
        mesh=mesh,
        scratch_types=[
            pltpu.SMEM((C,), jnp.int32),
            pltpu.SemaphoreType.DMA,
        ],
    )
    def run(in_hbm, idx_hbm, out_hbm, idx_smem, sem):
        pltpu.sync_copy(idx_hbm, idx_smem)
        base = lax.axis_index("core") * per_core

        def issue(i, carry):
            c = base + i
            src = idx_smem[c]

            def issue_b(b, carry2):
                pltpu.make_async_copy(
                    in_hbm.at[b, src], out_hbm.at[b, c], sem
                ).start()
                return carry2

            lax.fori_loop(0, B, issue_b, carry)
            return carry

        lax.fori_loop(0, per_core, issue, 0)

        def drain(i, carry):
            pltpu.make_async_copy(
                in_hbm.at[0, 0], out_hbm.at[0, 0], sem
            ).wait()
            return carry

        lax.fori_loop(0, per_core * B, drain, 0)

    return run(input, indices)


# speedup vs baseline: 9.6731x; 9.6731x over previous
import functools

import jax
import jax.numpy as jnp
from jax import lax
from jax.experimental import pallas as pl
from jax.experimental.pallas import tpu as pltpu
from jax.experimental.pallas import tpu_sc as plsc


def kernel(input, indices):
    B, C, H, W = input.shape
    SPLIT = 8
    D = (H * W) // SPLIT
    ROWS = B * C * SPLIT

    info = plsc.get_sparse_core_info()
    NW = info.num_cores * info.num_subcores
    per_w = ROWS // NW
    K = 16
    chunks = per_w // K

    q = jnp.arange(ROWS, dtype=jnp.int32)
    coarse, sub = q // SPLIT, q % SPLIT
    b, c = coarse // C, coarse % C
    gidx = (b * C + indices[c]) * SPLIT + sub

    x2d = input.reshape(ROWS, D)
    mesh = plsc.VectorSubcoreMesh(core_axis_name="c", subcore_axis_name="s")

    @functools.partial(
        pl.kernel,
        out_type=jax.ShapeDtypeStruct((ROWS, D), jnp.float32),
        mesh=mesh,
        scratch_types=[
            pltpu.VMEM((per_w,), jnp.int32),
            pltpu.VMEM((K, D), jnp.float32),
            pltpu.SemaphoreType.DMA,
        ],
    )
    def run(in_hbm, gidx_hbm, out_hbm, idx_v, rows_v, sem):
        wid = lax.axis_index("s") * info.num_cores + lax.axis_index("c")
        base = wid * per_w
        pltpu.sync_copy(gidx_hbm.at[pl.ds(base, per_w)], idx_v)

        def chunk(t, carry):
            off = t * K
            pltpu.make_async_copy(
                in_hbm.at[idx_v.at[pl.ds(off, K)]], rows_v, sem
            ).start()
            pltpu.make_async_copy(
                in_hbm.at[idx_v.at[pl.ds(off, K)]], rows_v, sem
            ).wait()
            pltpu.sync_copy(rows_v, out_hbm.at[pl.ds(base + off, K)])
            return carry

        lax.fori_loop(0, chunks, chunk, 0)

    out2d = run(x2d, gidx)
    return out2d.reshape(B, C, H, W)
